# Initial kernel scaffold; baseline (speedup 1.0000x reference)
import functools

import jax
import jax.numpy as jnp
from jax import lax
from jax.experimental import pallas as pl
from jax.experimental.pallas import tpu as pltpu
from jax.experimental.pallas import tpu_sc as plsc

NC = 2
NS = 16
NW = NC * NS
LANES = 16
IDXW = 128
CHUNK = 512
SUB = CHUNK // IDXW


def _emb_kernel(tok_hbm, comb_hbm, idx_hbm, cidx_hbm, out_hbm,
                idx_v, cidx_v, tok_v, comb_v, sem):
  n_rows, emb = out_hbm.shape
  rows_per_tile = n_rows // NW
  wid = lax.axis_index("s") * NC + lax.axis_index("c")
  row0 = wid * rows_per_tile

  @pl.loop(0, rows_per_tile, step=CHUNK)
  def _chunk(off):
    base = row0 + off
    pltpu.sync_copy(idx_hbm.at[pl.ds(base // IDXW, SUB)], idx_v)
    pltpu.sync_copy(cidx_hbm.at[pl.ds(base // IDXW, SUB)], cidx_v)

    copies = []
    for j in range(SUB):
      copies.append(pltpu.async_copy(
          tok_hbm.at[idx_v.at[j]], tok_v.at[pl.ds(j * IDXW, IDXW)], sem))
      copies.append(pltpu.async_copy(
          comb_hbm.at[cidx_v.at[j]], comb_v.at[pl.ds(j * IDXW, IDXW)], sem))
    for c in copies:
      c.wait()

    @pl.loop(0, CHUNK)
    def _row(i):
      for c in range(emb // LANES):
        sl = pl.ds(c * LANES, LANES)
        tok_v[i, sl] = tok_v[i, sl] + comb_v[i, sl]

    pltpu.sync_copy(tok_v, out_hbm.at[pl.ds(base, CHUNK)])


def kernel(x, segment_label, token_table, position_table, segment_table):
  batch, seq = x.shape
  emb = token_table.shape[1]
  n = batch * seq

  nseg = segment_table.shape[0]
  combined = (position_table[:seq, None, :]
              + segment_table[None, :, :]).reshape(seq * nseg, emb)

  idx = x.reshape(n // IDXW, IDXW).astype(jnp.int32)
  cidx = (jnp.arange(seq, dtype=jnp.int32)[None, :] * nseg
          + segment_label.astype(jnp.int32)).reshape(n // IDXW, IDXW)

  mesh = plsc.VectorSubcoreMesh(core_axis_name="c", subcore_axis_name="s",
                                num_cores=NC, num_subcores=NS)
  run = pl.kernel(
      _emb_kernel,
      out_type=jax.ShapeDtypeStruct((n, emb), jnp.float32),
      mesh=mesh,
      scratch_types=[
          pltpu.VMEM((SUB, IDXW), jnp.int32),
          pltpu.VMEM((SUB, IDXW), jnp.int32),
          pltpu.VMEM((CHUNK, emb), jnp.float32),
          pltpu.VMEM((CHUNK, emb), jnp.float32),
          pltpu.SemaphoreType.DMA,
      ],
  )
  out = run(token_table, combined, idx, cidx)
  return out.reshape(batch, seq, emb)


# speedup vs baseline: 2.2509x; 2.2509x over previous
import functools

import jax
import jax.numpy as jnp
from jax import lax
from jax.experimental import pallas as pl
from jax.experimental.pallas import tpu as pltpu
from jax.experimental.pallas import tpu_sc as plsc

NC = 2
NS = 16
NW = NC * NS
LANES = 16
IDXW = 128
CHUNK = 512
SUB = CHUNK // IDXW


def _emb_kernel(tok_hbm, comb_hbm, idx_hbm, cidx_hbm, out_hbm,
                idx_v, cidx_v, tok_v, comb_v, sem):
  n_rows, emb = out_hbm.shape
  rows_per_tile = n_rows // NW
  wid = lax.axis_index("s") * NC + lax.axis_index("c")
  row0 = wid * rows_per_tile

  @pl.loop(0, rows_per_tile, step=CHUNK)
  def _chunk(off):
    base = row0 + off
    pltpu.sync_copy(idx_hbm.at[pl.ds(base, CHUNK)], idx_v)
    pltpu.sync_copy(cidx_hbm.at[pl.ds(base, CHUNK)], cidx_v)

    copies = []
    for j in range(SUB):
      win = pl.ds(j * IDXW, IDXW)
      copies.append(pltpu.async_copy(
          tok_hbm.at[idx_v.at[win]], tok_v.at[win], sem))
      copies.append(pltpu.async_copy(
          comb_hbm.at[cidx_v.at[win]], comb_v.at[win], sem))
    for c in copies:
      c.wait()

    @pl.loop(0, CHUNK)
    def _row(i):
      for c in range(emb // LANES):
        sl = pl.ds(c * LANES, LANES)
        tok_v[i, sl] = tok_v[i, sl] + comb_v[i, sl]

    pltpu.sync_copy(tok_v, out_hbm.at[pl.ds(base, CHUNK)])


def kernel(x, segment_label, token_table, position_table, segment_table):
  batch, seq = x.shape
  emb = token_table.shape[1]
  n = batch * seq

  nseg = segment_table.shape[0]
  combined = (position_table[:seq, None, :]
              + segment_table[None, :, :]).reshape(seq * nseg, emb)

  idx = x.reshape(n).astype(jnp.int32)
  cidx = (jnp.arange(seq, dtype=jnp.int32)[None, :] * nseg
          + segment_label.astype(jnp.int32)).reshape(n)

  mesh = plsc.VectorSubcoreMesh(core_axis_name="c", subcore_axis_name="s",
                                num_cores=NC, num_subcores=NS)
  run = pl.kernel(
      _emb_kernel,
      out_type=jax.ShapeDtypeStruct((n, emb), jnp.float32),
      mesh=mesh,
      scratch_types=[
          pltpu.VMEM((CHUNK,), jnp.int32),
          pltpu.VMEM((CHUNK,), jnp.int32),
          pltpu.VMEM((CHUNK, emb), jnp.float32),
          pltpu.VMEM((CHUNK, emb), jnp.float32),
          pltpu.SemaphoreType.DMA,
      ],
      compiler_params=pltpu.CompilerParams(use_tc_tiling_on_sc=False),
  )
  out = run(token_table, combined, idx, cidx)
  return out.reshape(batch, seq, emb)


# speedup vs baseline: 2.3560x; 1.0467x over previous
import functools

import jax
import jax.numpy as jnp
from jax import lax
from jax.experimental import pallas as pl
from jax.experimental.pallas import tpu as pltpu
from jax.experimental.pallas import tpu_sc as plsc

NC = 2
NS = 16
NW = NC * NS
LANES = 16
IDXW = 128
CHUNK = 512
SUB = CHUNK // IDXW


def _emb_kernel(tok_hbm, comb_hbm, idx_hbm, cidx_hbm, out_hbm,
                idx_v, cidx_v, tok_v, comb_v, sem):
  n_rows, emb = out_hbm.shape
  rows_per_tile = n_rows // NW
  wid = lax.axis_index("s") * NC + lax.axis_index("c")
  row0 = wid * rows_per_tile

  @pl.loop(0, rows_per_tile, step=CHUNK)
  def _chunk(off):
    base = row0 + off
    pltpu.sync_copy(idx_hbm.at[pl.ds(base, CHUNK)], idx_v)
    pltpu.sync_copy(cidx_hbm.at[pl.ds(base, CHUNK)], cidx_v)

    copies = []
    for j in range(SUB):
      win = pl.ds(j * IDXW, IDXW)
      copies.append(pltpu.async_copy(
          tok_hbm.at[idx_v.at[win]], tok_v.at[win], sem))
    for c in copies:
      c.wait()
    copies = []
    for j in range(SUB):
      win = pl.ds(j * IDXW, IDXW)
      copies.append(pltpu.async_copy(
          comb_hbm.at[cidx_v.at[win]], tok_v.at[win], sem, add=True))
    for c in copies:
      c.wait()

    pltpu.sync_copy(tok_v, out_hbm.at[pl.ds(base, CHUNK)])


def kernel(x, segment_label, token_table, position_table, segment_table):
  batch, seq = x.shape
  emb = token_table.shape[1]
  n = batch * seq

  nseg = segment_table.shape[0]
  combined = (position_table[:seq, None, :]
              + segment_table[None, :, :]).reshape(seq * nseg, emb)

  idx = x.reshape(n).astype(jnp.int32)
  cidx = (jnp.arange(seq, dtype=jnp.int32)[None, :] * nseg
          + segment_label.astype(jnp.int32)).reshape(n)

  mesh = plsc.VectorSubcoreMesh(core_axis_name="c", subcore_axis_name="s",
                                num_cores=NC, num_subcores=NS)
  run = pl.kernel(
      _emb_kernel,
      out_type=jax.ShapeDtypeStruct((n, emb), jnp.float32),
      mesh=mesh,
      scratch_types=[
          pltpu.VMEM((CHUNK,), jnp.int32),
          pltpu.VMEM((CHUNK,), jnp.int32),
          pltpu.VMEM((CHUNK, emb), jnp.float32),
          pltpu.VMEM((CHUNK, emb), jnp.float32),
          pltpu.SemaphoreType.DMA,
      ],
      compiler_params=pltpu.CompilerParams(use_tc_tiling_on_sc=False),
  )
  out = run(token_table, combined, idx, cidx)
  return out.reshape(batch, seq, emb)
